# baseline (speedup 1.0000x reference)
import functools

import jax
import jax.numpy as jnp
from jax import lax
from jax.experimental import pallas as pl
from jax.experimental.pallas import tpu as pltpu
from jax.experimental.pallas import tpu_sc as plsc

N, E, D, H = 10000, 160000, 128, 32
NC, NS = 2, 16
BATCH = 128
EP = 163840
EDGES_PER_W = EP // NS
NBATCH = EDGES_PER_W // BATCH
NPAD = 10240
RPS = NPAD // NS
PAD_DST = N + 64
TCB = 400

_f32 = jnp.float32


def _sc_mesh():
    return plsc.VectorSubcoreMesh(core_axis_name="c", subcore_axis_name="s",
                                  num_cores=NC, num_subcores=NS)


def _sc_sums(table, edges, zeros128):
    HB = NBATCH // 2

    @functools.partial(
        pl.kernel,
        out_type=jax.ShapeDtypeStruct((2, NPAD, 128), _f32),
        mesh=_sc_mesh(),
        scratch_types=[
            pltpu.VMEM((NBATCH // 2, BATCH), jnp.int32),
            pltpu.VMEM((NBATCH // 2, BATCH), jnp.int32),
            pltpu.VMEM((BATCH, 128), _f32),
            pltpu.VMEM((BATCH, 128), _f32),
            pltpu.VMEM_SHARED((NPAD, 128), _f32),
            pltpu.SemaphoreType.DMA,
            pltpu.SemaphoreType.DMA,
            pltpu.SemaphoreType.DMA,
            pltpu.SemaphoreType.DMA,
        ],
    )
    def k(table_h, edges_h, zeros128_h, out_s,
          src_half, dst_half, rows0, rows1, acc, gA, gB, sA, sB):
        cid = lax.axis_index("c")
        sid = lax.axis_index("s")
        r0 = sid * RPS
        pltpu.sync_copy(zeros128_h.at[pl.ds(r0, RPS)], acc.at[pl.ds(r0, RPS)])
        plsc.subcore_barrier()

        def wait_g(buf, sem):
            pltpu.make_async_copy(table_h.at[src_half.at[0]], buf, sem).wait()

        def wait_s(buf, sem):
            pltpu.make_async_copy(buf, acc.at[dst_half.at[0]], sem).wait()

        for half in range(2):
            pltpu.sync_copy(edges_h.at[cid, 0, sid, pl.ds(half * HB, HB)], src_half)
            pltpu.sync_copy(edges_h.at[cid, 1, sid, pl.ds(half * HB, HB)], dst_half)
            pltpu.async_copy(table_h.at[src_half.at[0]], rows0, gA)
            pltpu.async_copy(table_h.at[src_half.at[1]], rows1, gB)

            def step(i, carry):
                j0 = 2 * i
                wait_g(rows0, gA)
                pltpu.async_copy(rows0, acc.at[dst_half.at[j0]], sA, add=True)
                wait_g(rows1, gB)
                pltpu.async_copy(rows1, acc.at[dst_half.at[j0 + 1]], sB, add=True)

                @pl.when(j0 + 2 < HB)
                def _():
                    wait_s(rows0, sA)
                    pltpu.async_copy(table_h.at[src_half.at[j0 + 2]], rows0, gA)
                    wait_s(rows1, sB)
                    pltpu.async_copy(table_h.at[src_half.at[j0 + 3]], rows1, gB)

                return carry

            lax.fori_loop(0, HB // 2, step, 0)
            wait_s(rows0, sA)
            wait_s(rows1, sB)
        plsc.subcore_barrier()
        pltpu.sync_copy(acc.at[pl.ds(r0, RPS)], out_s.at[cid, pl.ds(r0, RPS)])

    return k(table, edges, zeros128)


def _means(s, w):
    cp = jnp.maximum(s[0, :, 127:], 1.0)
    cn = jnp.maximum(s[1, :, 127:], 1.0)
    gp = s[0, :, :w] / cp
    gn = s[1, :, :w] / cn
    return gp, gn


def _with_ones_col(t):
    r = t.shape[0]
    return jnp.concatenate(
        [t, jnp.zeros((r, 127 - t.shape[1]), _f32), jnp.ones((r, 1), _f32)],
        axis=1)


def _tc_proj(x, wtop, wbot):
    def body(x_ref, wt_ref, wb_ref, o1_ref, o2_ref):
        xb = x_ref[...]
        o1_ref[...] = _with_ones_col(
            jnp.dot(xb, wt_ref[...], preferred_element_type=_f32))
        o2_ref[...] = jnp.dot(xb, wb_ref[...], preferred_element_type=_f32)

    return pl.pallas_call(
        body,
        grid=(N // TCB,),
        in_specs=[pl.BlockSpec((TCB, D), lambda i: (i, 0)),
                  pl.BlockSpec((D, 64), lambda i: (0, 0)),
                  pl.BlockSpec((D, 64), lambda i: (0, 0))],
        out_specs=(pl.BlockSpec((TCB, 128), lambda i: (i, 0)),
                   pl.BlockSpec((TCB, 64), lambda i: (i, 0))),
        out_shape=(jax.ShapeDtypeStruct((N, 128), _f32),
                   jax.ShapeDtypeStruct((N, 64), _f32)),
    )(x, wtop, wbot)


def _tc_layer0(s0, xself, b0):
    def body(s_ref, p_ref, b_ref, o_ref):
        gp, gn = _means(s_ref[...], 64)
        z = jnp.concatenate([gp[:, :32], gn[:, 32:]], axis=1) + p_ref[...] + b_ref[...]
        o_ref[...] = _with_ones_col(jnp.tanh(z))

    return pl.pallas_call(
        body,
        grid=(N // TCB,),
        in_specs=[pl.BlockSpec((2, TCB, 128), lambda i: (0, i, 0)),
                  pl.BlockSpec((TCB, 64), lambda i: (i, 0)),
                  pl.BlockSpec((1, 64), lambda i: (0, 0))],
        out_specs=pl.BlockSpec((TCB, 128), lambda i: (i, 0)),
        out_shape=jax.ShapeDtypeStruct((N, 128), _f32),
    )(s0, xself, b0)


def _tc_deep(s, h, wa, wb, wc, b, pad_out):
    def body(s_ref, h_ref, wa_ref, wb_ref, wc_ref, b_ref, o_ref):
        gp, gn = _means(s_ref[...], 64)
        z = (jnp.dot(gp, wa_ref[...], preferred_element_type=_f32)
             + jnp.dot(gn, wb_ref[...], preferred_element_type=_f32)
             + jnp.dot(h_ref[...][:, :64], wc_ref[...],
                       preferred_element_type=_f32)
             + b_ref[...])
        t = jnp.tanh(z)
        o_ref[...] = _with_ones_col(t) if pad_out else t

    ocols = 128 if pad_out else 64
    return pl.pallas_call(
        body,
        grid=(N // TCB,),
        in_specs=[pl.BlockSpec((2, TCB, 128), lambda i: (0, i, 0)),
                  pl.BlockSpec((TCB, 128), lambda i: (i, 0)),
                  pl.BlockSpec((64, 64), lambda i: (0, 0)),
                  pl.BlockSpec((64, 64), lambda i: (0, 0)),
                  pl.BlockSpec((64, 64), lambda i: (0, 0)),
                  pl.BlockSpec((1, 64), lambda i: (0, 0))],
        out_specs=pl.BlockSpec((TCB, ocols), lambda i: (i, 0)),
        out_shape=jax.ShapeDtypeStruct((N, ocols), _f32),
    )(s, h, wa, wb, wc, b)


def _pad_edges(ei):
    pad = EP - E
    src = jnp.concatenate([ei[0], jnp.zeros((pad,), jnp.int32)])
    dst = jnp.concatenate([ei[1], jnp.full((pad,), PAD_DST, jnp.int32)])
    return jnp.stack([src.reshape(NS, NBATCH, BATCH),
                      dst.reshape(NS, NBATCH, BATCH)])


def _deep_weights(wp, wn, bp, bn):
    blk = lambda w, i: w[i * H:(i + 1) * H]
    wa = jnp.concatenate([jnp.concatenate([blk(wp, 0), blk(wn, 3)], axis=1),
                          jnp.concatenate([blk(wp, 2), blk(wn, 1)], axis=1)], axis=0)
    wb = jnp.concatenate([jnp.concatenate([blk(wp, 3), blk(wn, 0)], axis=1),
                          jnp.concatenate([blk(wp, 1), blk(wn, 2)], axis=1)], axis=0)
    wc = jnp.concatenate(
        [jnp.concatenate([blk(wp, 4) + 0.5 * blk(wp, 6),
                          blk(wn, 5) + 0.5 * blk(wn, 6)], axis=1),
         jnp.concatenate([blk(wp, 5) + 0.5 * blk(wp, 6),
                          blk(wn, 4) + 0.5 * blk(wn, 6)], axis=1)], axis=0)
    b = jnp.concatenate([bp, bn]).reshape(1, 64)
    return wa, wb, wc, b


def kernel(x, W_pos_base, b_pos_base, W_neg_base, b_neg_base,
           W_pos_d1, b_pos_d1, W_pos_d2, b_pos_d2,
           W_neg_d1, b_neg_d1, W_neg_d2, b_neg_d2,
           pos_edge_index, neg_edge_index):
    edges = jnp.stack([_pad_edges(pos_edge_index), _pad_edges(neg_edge_index)])
    zeros128 = jnp.zeros((NPAD, 128), _f32)

    wtop = jnp.concatenate([W_pos_base[:D], W_neg_base[:D]], axis=1)
    wbot = jnp.concatenate([W_pos_base[D:], W_neg_base[D:]], axis=1)
    b0 = jnp.concatenate([b_pos_base, b_neg_base]).reshape(1, 64)

    table0, xself = _tc_proj(x, wtop, wbot)
    s0 = _sc_sums(table0, edges, zeros128)
    h = _tc_layer0(s0, xself, b0)
    for li, (wp, bp, wn, bn) in enumerate(
            ((W_pos_d1, b_pos_d1, W_neg_d1, b_neg_d1),
             (W_pos_d2, b_pos_d2, W_neg_d2, b_neg_d2))):
        wa, wb, wc, b = _deep_weights(wp, wn, bp, bn)
        s = _sc_sums(h, edges, zeros128)
        h = _tc_deep(s, h, wa, wb, wc, b, pad_out=(li == 0))
    return h


# speedup vs baseline: 1.0905x; 1.0905x over previous
import functools

import jax
import jax.numpy as jnp
from jax import lax
from jax.experimental import pallas as pl
from jax.experimental.pallas import tpu as pltpu
from jax.experimental.pallas import tpu_sc as plsc

N, E, D, H = 10000, 160000, 128, 32
NC, NS = 2, 16
BATCH = 128
EP = 163840
EDGES_PER_W = EP // NS
NBATCH = EDGES_PER_W // BATCH
NPAD = 10240
RPS = NPAD // NS
PAD_DST = N + 64
TCB = 2000

_f32 = jnp.float32


def _sc_mesh():
    return plsc.VectorSubcoreMesh(core_axis_name="c", subcore_axis_name="s",
                                  num_cores=NC, num_subcores=NS)


def _sc_sums(table, edges, zeros128):
    HB = NBATCH // 2

    @functools.partial(
        pl.kernel,
        out_type=jax.ShapeDtypeStruct((2, NPAD, 128), _f32),
        mesh=_sc_mesh(),
        scratch_types=[
            pltpu.VMEM((NBATCH // 2, BATCH), jnp.int32),
            pltpu.VMEM((NBATCH // 2, BATCH), jnp.int32),
            pltpu.VMEM((BATCH, 128), _f32),
            pltpu.VMEM((BATCH, 128), _f32),
            pltpu.VMEM_SHARED((NPAD, 128), _f32),
            pltpu.SemaphoreType.DMA,
            pltpu.SemaphoreType.DMA,
        ],
    )
    def k(table_h, edges_h, zeros128_h, out_s,
          src_half, dst_half, rows0, rows1, acc, semA, semB):
        cid = lax.axis_index("c")
        sid = lax.axis_index("s")
        r0 = sid * RPS
        pltpu.sync_copy(zeros128_h.at[pl.ds(r0, RPS)], acc.at[pl.ds(r0, RPS)])
        plsc.subcore_barrier()

        def wait_for(buf, sem):
            pltpu.make_async_copy(table_h.at[src_half.at[0]], buf, sem).wait()

        for half in range(2):
            pltpu.sync_copy(edges_h.at[cid, 0, sid, pl.ds(half * HB, HB)], src_half)
            pltpu.sync_copy(edges_h.at[cid, 1, sid, pl.ds(half * HB, HB)], dst_half)
            pltpu.async_copy(table_h.at[src_half.at[0]], rows0, semA)

            def step(i, carry):
                j0 = 2 * i
                pltpu.async_copy(table_h.at[src_half.at[j0 + 1]], rows1, semB)
                wait_for(rows0, semA)
                pltpu.sync_copy(rows0, acc.at[dst_half.at[j0]], add=True)

                @pl.when(j0 + 2 < HB)
                def _():
                    pltpu.async_copy(table_h.at[src_half.at[j0 + 2]], rows0, semA)

                wait_for(rows1, semB)
                pltpu.sync_copy(rows1, acc.at[dst_half.at[j0 + 1]], add=True)
                return carry

            lax.fori_loop(0, HB // 2, step, 0)
        plsc.subcore_barrier()
        pltpu.sync_copy(acc.at[pl.ds(r0, RPS)], out_s.at[cid, pl.ds(r0, RPS)])

    return k(table, edges, zeros128)


def _means(s, w):
    cp = jnp.maximum(s[0, :, 127:], 1.0)
    cn = jnp.maximum(s[1, :, 127:], 1.0)
    gp = s[0, :, :w] / cp
    gn = s[1, :, :w] / cn
    return gp, gn


def _with_ones_col(t):
    r = t.shape[0]
    return jnp.concatenate(
        [t, jnp.zeros((r, 127 - t.shape[1]), _f32), jnp.ones((r, 1), _f32)],
        axis=1)


def _tc_proj(x, wtop, wbot):
    def body(x_ref, wt_ref, wb_ref, o1_ref, o2_ref):
        xb = x_ref[...]
        o1_ref[...] = _with_ones_col(
            jnp.dot(xb, wt_ref[...], preferred_element_type=_f32))
        o2_ref[...] = jnp.dot(xb, wb_ref[...], preferred_element_type=_f32)

    return pl.pallas_call(
        body,
        grid=(N // TCB,),
        in_specs=[pl.BlockSpec((TCB, D), lambda i: (i, 0)),
                  pl.BlockSpec((D, 64), lambda i: (0, 0)),
                  pl.BlockSpec((D, 64), lambda i: (0, 0))],
        out_specs=(pl.BlockSpec((TCB, 128), lambda i: (i, 0)),
                   pl.BlockSpec((TCB, 64), lambda i: (i, 0))),
        out_shape=(jax.ShapeDtypeStruct((N, 128), _f32),
                   jax.ShapeDtypeStruct((N, 64), _f32)),
    )(x, wtop, wbot)


def _tc_layer0(s0, xself, b0):
    def body(s_ref, p_ref, b_ref, o_ref):
        gp, gn = _means(s_ref[...], 64)
        z = jnp.concatenate([gp[:, :32], gn[:, 32:]], axis=1) + p_ref[...] + b_ref[...]
        o_ref[...] = _with_ones_col(jnp.tanh(z))

    return pl.pallas_call(
        body,
        grid=(N // TCB,),
        in_specs=[pl.BlockSpec((2, TCB, 128), lambda i: (0, i, 0)),
                  pl.BlockSpec((TCB, 64), lambda i: (i, 0)),
                  pl.BlockSpec((1, 64), lambda i: (0, 0))],
        out_specs=pl.BlockSpec((TCB, 128), lambda i: (i, 0)),
        out_shape=jax.ShapeDtypeStruct((N, 128), _f32),
    )(s0, xself, b0)


def _tc_deep(s, h, wa, wb, wc, b, pad_out):
    def body(s_ref, h_ref, wa_ref, wb_ref, wc_ref, b_ref, o_ref):
        gp, gn = _means(s_ref[...], 64)
        z = (jnp.dot(gp, wa_ref[...], preferred_element_type=_f32)
             + jnp.dot(gn, wb_ref[...], preferred_element_type=_f32)
             + jnp.dot(h_ref[...][:, :64], wc_ref[...],
                       preferred_element_type=_f32)
             + b_ref[...])
        t = jnp.tanh(z)
        o_ref[...] = _with_ones_col(t) if pad_out else t

    ocols = 128 if pad_out else 64
    return pl.pallas_call(
        body,
        grid=(N // TCB,),
        in_specs=[pl.BlockSpec((2, TCB, 128), lambda i: (0, i, 0)),
                  pl.BlockSpec((TCB, 128), lambda i: (i, 0)),
                  pl.BlockSpec((64, 64), lambda i: (0, 0)),
                  pl.BlockSpec((64, 64), lambda i: (0, 0)),
                  pl.BlockSpec((64, 64), lambda i: (0, 0)),
                  pl.BlockSpec((1, 64), lambda i: (0, 0))],
        out_specs=pl.BlockSpec((TCB, ocols), lambda i: (i, 0)),
        out_shape=jax.ShapeDtypeStruct((N, ocols), _f32),
    )(s, h, wa, wb, wc, b)


def _pad_edges(ei):
    pad = EP - E
    src = jnp.concatenate([ei[0], jnp.zeros((pad,), jnp.int32)])
    dst = jnp.concatenate([ei[1], jnp.full((pad,), PAD_DST, jnp.int32)])
    return jnp.stack([src.reshape(NS, NBATCH, BATCH),
                      dst.reshape(NS, NBATCH, BATCH)])


def _deep_weights(wp, wn, bp, bn):
    blk = lambda w, i: w[i * H:(i + 1) * H]
    wa = jnp.concatenate([jnp.concatenate([blk(wp, 0), blk(wn, 3)], axis=1),
                          jnp.concatenate([blk(wp, 2), blk(wn, 1)], axis=1)], axis=0)
    wb = jnp.concatenate([jnp.concatenate([blk(wp, 3), blk(wn, 0)], axis=1),
                          jnp.concatenate([blk(wp, 1), blk(wn, 2)], axis=1)], axis=0)
    wc = jnp.concatenate(
        [jnp.concatenate([blk(wp, 4) + 0.5 * blk(wp, 6),
                          blk(wn, 5) + 0.5 * blk(wn, 6)], axis=1),
         jnp.concatenate([blk(wp, 5) + 0.5 * blk(wp, 6),
                          blk(wn, 4) + 0.5 * blk(wn, 6)], axis=1)], axis=0)
    b = jnp.concatenate([bp, bn]).reshape(1, 64)
    return wa, wb, wc, b


def kernel(x, W_pos_base, b_pos_base, W_neg_base, b_neg_base,
           W_pos_d1, b_pos_d1, W_pos_d2, b_pos_d2,
           W_neg_d1, b_neg_d1, W_neg_d2, b_neg_d2,
           pos_edge_index, neg_edge_index):
    edges = jnp.stack([_pad_edges(pos_edge_index), _pad_edges(neg_edge_index)])
    zeros128 = jnp.zeros((NPAD, 128), _f32)

    wtop = jnp.concatenate([W_pos_base[:D], W_neg_base[:D]], axis=1)
    wbot = jnp.concatenate([W_pos_base[D:], W_neg_base[D:]], axis=1)
    b0 = jnp.concatenate([b_pos_base, b_neg_base]).reshape(1, 64)

    table0, xself = _tc_proj(x, wtop, wbot)
    s0 = _sc_sums(table0, edges, zeros128)
    h = _tc_layer0(s0, xself, b0)
    for li, (wp, bp, wn, bn) in enumerate(
            ((W_pos_d1, b_pos_d1, W_neg_d1, b_neg_d1),
             (W_pos_d2, b_pos_d2, W_neg_d2, b_neg_d2))):
        wa, wb, wc, b = _deep_weights(wp, wn, bp, bn)
        s = _sc_sums(h, edges, zeros128)
        h = _tc_deep(s, h, wa, wb, wc, b, pad_out=(li == 0))
    return h


# speedup vs baseline: 1.6956x; 1.5548x over previous
import functools

import jax
import jax.numpy as jnp
from jax import lax
from jax.experimental import pallas as pl
from jax.experimental.pallas import tpu as pltpu
from jax.experimental.pallas import tpu_sc as plsc

N, E, D, H = 10000, 160000, 128, 32
NC, NS = 2, 16
BATCH = 128
EP = 163840
EDGES_PER_W = EP // NS
NBATCH = EDGES_PER_W // BATCH
NPAD = 10240
RPS = NPAD // NS
PAD_DST = N + 64
TCB = 2000
W0 = 80

_f32 = jnp.float32


def _sc_mesh():
    return plsc.VectorSubcoreMesh(core_axis_name="c", subcore_axis_name="s",
                                  num_cores=NC, num_subcores=NS)


def _sc_sums(table, edges, zeros, w):
    HB = NBATCH // 2

    @functools.partial(
        pl.kernel,
        out_type=jax.ShapeDtypeStruct((2, NPAD, w), _f32),
        mesh=_sc_mesh(),
        scratch_types=[
            pltpu.VMEM((HB, BATCH), jnp.int32),
            pltpu.VMEM((HB, BATCH), jnp.int32),
            pltpu.VMEM((BATCH, w), _f32),
            pltpu.VMEM((BATCH, w), _f32),
            pltpu.VMEM_SHARED((NPAD, w), _f32),
            pltpu.SemaphoreType.DMA,
            pltpu.SemaphoreType.DMA,
        ],
        compiler_params=pltpu.CompilerParams(use_tc_tiling_on_sc=False),
    )
    def k(table_h, edges_h, zeros_h, out_s,
          src_half, dst_half, rows0, rows1, acc, semA, semB):
        cid = lax.axis_index("c")
        sid = lax.axis_index("s")
        r0 = sid * RPS
        pltpu.sync_copy(zeros_h.at[pl.ds(r0, RPS)], acc.at[pl.ds(r0, RPS)])
        plsc.subcore_barrier()

        def wait_for(buf, sem):
            pltpu.make_async_copy(table_h.at[src_half.at[0]], buf, sem).wait()

        for half in range(2):
            pltpu.sync_copy(edges_h.at[cid, 0, sid, pl.ds(half * HB, HB)], src_half)
            pltpu.sync_copy(edges_h.at[cid, 1, sid, pl.ds(half * HB, HB)], dst_half)
            pltpu.async_copy(table_h.at[src_half.at[0]], rows0, semA)

            def step(i, carry):
                j0 = 2 * i
                pltpu.async_copy(table_h.at[src_half.at[j0 + 1]], rows1, semB)
                wait_for(rows0, semA)
                pltpu.sync_copy(rows0, acc.at[dst_half.at[j0]], add=True)

                @pl.when(j0 + 2 < HB)
                def _():
                    pltpu.async_copy(table_h.at[src_half.at[j0 + 2]], rows0, semA)

                wait_for(rows1, semB)
                pltpu.sync_copy(rows1, acc.at[dst_half.at[j0 + 1]], add=True)
                return carry

            lax.fori_loop(0, HB // 2, step, 0)
        plsc.subcore_barrier()
        pltpu.sync_copy(acc.at[pl.ds(r0, RPS)], out_s.at[cid, pl.ds(r0, RPS)])

    return k(table, edges, zeros)


def _inv_counts(s0):
    cp = jnp.maximum(s0[0, :, W0 - 1:], 1.0)
    cn = jnp.maximum(s0[1, :, W0 - 1:], 1.0)
    return cp, cn


def _tc_proj(x, wtop, wbot):
    def body(x_ref, wt_ref, wb_ref, o1_ref, o2_ref):
        xb = x_ref[...]
        p = jnp.dot(xb, wt_ref[...], preferred_element_type=_f32)
        r = p.shape[0]
        o1_ref[...] = jnp.concatenate(
            [p, jnp.zeros((r, W0 - 65), _f32), jnp.ones((r, 1), _f32)], axis=1)
        o2_ref[...] = jnp.dot(xb, wb_ref[...], preferred_element_type=_f32)

    return pl.pallas_call(
        body,
        grid=(N // TCB,),
        in_specs=[pl.BlockSpec((TCB, D), lambda i: (i, 0)),
                  pl.BlockSpec((D, 64), lambda i: (0, 0)),
                  pl.BlockSpec((D, 64), lambda i: (0, 0))],
        out_specs=(pl.BlockSpec((TCB, W0), lambda i: (i, 0)),
                   pl.BlockSpec((TCB, 64), lambda i: (i, 0))),
        out_shape=(jax.ShapeDtypeStruct((N, W0), _f32),
                   jax.ShapeDtypeStruct((N, 64), _f32)),
    )(x, wtop, wbot)


def _tc_layer0(s0, xself, b0):
    def body(s_ref, p_ref, b_ref, o_ref):
        s0b = s_ref[...]
        cp, cn = _inv_counts(s0b)
        gp = s0b[0, :, :64] / cp
        gn = s0b[1, :, :64] / cn
        z = jnp.concatenate([gp[:, :32], gn[:, 32:]], axis=1) + p_ref[...] + b_ref[...]
        o_ref[...] = jnp.tanh(z)

    return pl.pallas_call(
        body,
        grid=(N // TCB,),
        in_specs=[pl.BlockSpec((2, TCB, W0), lambda i: (0, i, 0)),
                  pl.BlockSpec((TCB, 64), lambda i: (i, 0)),
                  pl.BlockSpec((1, 64), lambda i: (0, 0))],
        out_specs=pl.BlockSpec((TCB, 64), lambda i: (i, 0)),
        out_shape=jax.ShapeDtypeStruct((N, 64), _f32),
    )(s0, xself, b0)


def _tc_deep(s, s0, h, wa, wb, wc, b):
    def body(s_ref, s0_ref, h_ref, wa_ref, wb_ref, wc_ref, b_ref, o_ref):
        cp, cn = _inv_counts(s0_ref[...])
        sb = s_ref[...]
        gp = sb[0] / cp
        gn = sb[1] / cn
        z = (jnp.dot(gp, wa_ref[...], preferred_element_type=_f32)
             + jnp.dot(gn, wb_ref[...], preferred_element_type=_f32)
             + jnp.dot(h_ref[...], wc_ref[...], preferred_element_type=_f32)
             + b_ref[...])
        o_ref[...] = jnp.tanh(z)

    return pl.pallas_call(
        body,
        grid=(N // TCB,),
        in_specs=[pl.BlockSpec((2, TCB, 64), lambda i: (0, i, 0)),
                  pl.BlockSpec((2, TCB, W0), lambda i: (0, i, 0)),
                  pl.BlockSpec((TCB, 64), lambda i: (i, 0)),
                  pl.BlockSpec((64, 64), lambda i: (0, 0)),
                  pl.BlockSpec((64, 64), lambda i: (0, 0)),
                  pl.BlockSpec((64, 64), lambda i: (0, 0)),
                  pl.BlockSpec((1, 64), lambda i: (0, 0))],
        out_specs=pl.BlockSpec((TCB, 64), lambda i: (i, 0)),
        out_shape=jax.ShapeDtypeStruct((N, 64), _f32),
    )(s, s0, h, wa, wb, wc, b)


def _pad_edges(ei):
    pad = EP - E
    src = jnp.concatenate([ei[0], jnp.zeros((pad,), jnp.int32)])
    dst = jnp.concatenate([ei[1], jnp.full((pad,), PAD_DST, jnp.int32)])
    return jnp.stack([src.reshape(NS, NBATCH, BATCH),
                      dst.reshape(NS, NBATCH, BATCH)])


def _deep_weights(wp, wn, bp, bn):
    blk = lambda w, i: w[i * H:(i + 1) * H]
    wa = jnp.concatenate([jnp.concatenate([blk(wp, 0), blk(wn, 3)], axis=1),
                          jnp.concatenate([blk(wp, 2), blk(wn, 1)], axis=1)], axis=0)
    wb = jnp.concatenate([jnp.concatenate([blk(wp, 3), blk(wn, 0)], axis=1),
                          jnp.concatenate([blk(wp, 1), blk(wn, 2)], axis=1)], axis=0)
    wc = jnp.concatenate(
        [jnp.concatenate([blk(wp, 4) + 0.5 * blk(wp, 6),
                          blk(wn, 5) + 0.5 * blk(wn, 6)], axis=1),
         jnp.concatenate([blk(wp, 5) + 0.5 * blk(wp, 6),
                          blk(wn, 4) + 0.5 * blk(wn, 6)], axis=1)], axis=0)
    b = jnp.concatenate([bp, bn]).reshape(1, 64)
    return wa, wb, wc, b


def kernel(x, W_pos_base, b_pos_base, W_neg_base, b_neg_base,
           W_pos_d1, b_pos_d1, W_pos_d2, b_pos_d2,
           W_neg_d1, b_neg_d1, W_neg_d2, b_neg_d2,
           pos_edge_index, neg_edge_index):
    edges = jnp.stack([_pad_edges(pos_edge_index), _pad_edges(neg_edge_index)])
    zeros80 = jnp.zeros((NPAD, W0), _f32)
    zeros64 = jnp.zeros((NPAD, 64), _f32)

    wtop = jnp.concatenate([W_pos_base[:D], W_neg_base[:D]], axis=1)
    wbot = jnp.concatenate([W_pos_base[D:], W_neg_base[D:]], axis=1)
    b0 = jnp.concatenate([b_pos_base, b_neg_base]).reshape(1, 64)

    table0, xself = _tc_proj(x, wtop, wbot)
    s0 = _sc_sums(table0, edges, zeros80, W0)
    h = _tc_layer0(s0, xself, b0)
    for (wp, bp, wn, bn) in ((W_pos_d1, b_pos_d1, W_neg_d1, b_neg_d1),
                             (W_pos_d2, b_pos_d2, W_neg_d2, b_neg_d2)):
        wa, wb, wc, b = _deep_weights(wp, wn, bp, bn)
        s = _sc_sums(h, edges, zeros64, 64)
        h = _tc_deep(s, s0, h, wa, wb, wc, b)
    return h


# speedup vs baseline: 1.7077x; 1.0071x over previous
import functools

import jax
import jax.numpy as jnp
from jax import lax
from jax.experimental import pallas as pl
from jax.experimental.pallas import tpu as pltpu
from jax.experimental.pallas import tpu_sc as plsc

N, E, D, H = 10000, 160000, 128, 32
NC, NS = 2, 16
BATCH = 128
EP = 163840
EDGES_PER_W = EP // NS
NBATCH = EDGES_PER_W // BATCH
NPAD = 10240
RPS = NPAD // NS
PAD_DST = N + 64
TCB = 10000
W0 = 72

_f32 = jnp.float32


def _sc_mesh():
    return plsc.VectorSubcoreMesh(core_axis_name="c", subcore_axis_name="s",
                                  num_cores=NC, num_subcores=NS)


def _sc_sums(table, edges, zeros, w):
    HB = NBATCH // 2

    @functools.partial(
        pl.kernel,
        out_type=jax.ShapeDtypeStruct((2, NPAD, w), _f32),
        mesh=_sc_mesh(),
        scratch_types=[
            pltpu.VMEM((HB, BATCH), jnp.int32),
            pltpu.VMEM((HB, BATCH), jnp.int32),
            pltpu.VMEM((BATCH, w), _f32),
            pltpu.VMEM((BATCH, w), _f32),
            pltpu.VMEM_SHARED((NPAD, w), _f32),
            pltpu.SemaphoreType.DMA,
            pltpu.SemaphoreType.DMA,
        ],
        compiler_params=pltpu.CompilerParams(use_tc_tiling_on_sc=False),
    )
    def k(table_h, edges_h, zeros_h, out_s,
          src_half, dst_half, rows0, rows1, acc, semA, semB):
        cid = lax.axis_index("c")
        sid = lax.axis_index("s")
        r0 = sid * RPS
        pltpu.sync_copy(zeros_h.at[pl.ds(r0, RPS)], acc.at[pl.ds(r0, RPS)])
        plsc.subcore_barrier()

        def wait_for(buf, sem):
            pltpu.make_async_copy(table_h.at[src_half.at[0]], buf, sem).wait()

        for half in range(2):
            pltpu.sync_copy(edges_h.at[cid, 0, sid, pl.ds(half * HB, HB)], src_half)
            pltpu.sync_copy(edges_h.at[cid, 1, sid, pl.ds(half * HB, HB)], dst_half)
            pltpu.async_copy(table_h.at[src_half.at[0]], rows0, semA)

            def step(i, carry):
                j0 = 2 * i
                pltpu.async_copy(table_h.at[src_half.at[j0 + 1]], rows1, semB)
                wait_for(rows0, semA)
                pltpu.sync_copy(rows0, acc.at[dst_half.at[j0]], add=True)

                @pl.when(j0 + 2 < HB)
                def _():
                    pltpu.async_copy(table_h.at[src_half.at[j0 + 2]], rows0, semA)

                wait_for(rows1, semB)
                pltpu.sync_copy(rows1, acc.at[dst_half.at[j0 + 1]], add=True)
                return carry

            lax.fori_loop(0, HB // 2, step, 0)
        plsc.subcore_barrier()
        pltpu.sync_copy(acc.at[pl.ds(r0, RPS)], out_s.at[cid, pl.ds(r0, RPS)])

    return k(table, edges, zeros)


def _inv_counts(s0):
    cp = jnp.maximum(s0[0, :, W0 - 1:], 1.0)
    cn = jnp.maximum(s0[1, :, W0 - 1:], 1.0)
    return cp, cn


def _tc_proj(x, wtop, wbot):
    def body(x_ref, wt_ref, wb_ref, o1_ref, o2_ref):
        xb = x_ref[...]
        p = jnp.dot(xb, wt_ref[...], preferred_element_type=_f32)
        r = p.shape[0]
        o1_ref[...] = jnp.concatenate(
            [p, jnp.zeros((r, W0 - 65), _f32), jnp.ones((r, 1), _f32)], axis=1)
        o2_ref[...] = jnp.dot(xb, wb_ref[...], preferred_element_type=_f32)

    return pl.pallas_call(
        body,
        grid=(N // TCB,),
        in_specs=[pl.BlockSpec((TCB, D), lambda i: (i, 0)),
                  pl.BlockSpec((D, 64), lambda i: (0, 0)),
                  pl.BlockSpec((D, 64), lambda i: (0, 0))],
        out_specs=(pl.BlockSpec((TCB, W0), lambda i: (i, 0)),
                   pl.BlockSpec((TCB, 64), lambda i: (i, 0))),
        out_shape=(jax.ShapeDtypeStruct((N, W0), _f32),
                   jax.ShapeDtypeStruct((N, 64), _f32)),
    )(x, wtop, wbot)


def _tc_layer0(s0, xself, b0):
    def body(s_ref, p_ref, b_ref, o_ref):
        s0b = s_ref[...]
        cp, cn = _inv_counts(s0b)
        gp = s0b[0, :, :64] / cp
        gn = s0b[1, :, :64] / cn
        z = jnp.concatenate([gp[:, :32], gn[:, 32:]], axis=1) + p_ref[...] + b_ref[...]
        o_ref[...] = jnp.tanh(z)

    return pl.pallas_call(
        body,
        grid=(N // TCB,),
        in_specs=[pl.BlockSpec((2, TCB, W0), lambda i: (0, i, 0)),
                  pl.BlockSpec((TCB, 64), lambda i: (i, 0)),
                  pl.BlockSpec((1, 64), lambda i: (0, 0))],
        out_specs=pl.BlockSpec((TCB, 64), lambda i: (i, 0)),
        out_shape=jax.ShapeDtypeStruct((N, 64), _f32),
    )(s0, xself, b0)


def _tc_deep(s, s0, h, wa, wb, wc, b):
    def body(s_ref, s0_ref, h_ref, wa_ref, wb_ref, wc_ref, b_ref, o_ref):
        cp, cn = _inv_counts(s0_ref[...])
        sb = s_ref[...]
        gp = sb[0] / cp
        gn = sb[1] / cn
        z = (jnp.dot(gp, wa_ref[...], preferred_element_type=_f32)
             + jnp.dot(gn, wb_ref[...], preferred_element_type=_f32)
             + jnp.dot(h_ref[...], wc_ref[...], preferred_element_type=_f32)
             + b_ref[...])
        o_ref[...] = jnp.tanh(z)

    return pl.pallas_call(
        body,
        grid=(N // TCB,),
        in_specs=[pl.BlockSpec((2, TCB, 64), lambda i: (0, i, 0)),
                  pl.BlockSpec((2, TCB, W0), lambda i: (0, i, 0)),
                  pl.BlockSpec((TCB, 64), lambda i: (i, 0)),
                  pl.BlockSpec((64, 64), lambda i: (0, 0)),
                  pl.BlockSpec((64, 64), lambda i: (0, 0)),
                  pl.BlockSpec((64, 64), lambda i: (0, 0)),
                  pl.BlockSpec((1, 64), lambda i: (0, 0))],
        out_specs=pl.BlockSpec((TCB, 64), lambda i: (i, 0)),
        out_shape=jax.ShapeDtypeStruct((N, 64), _f32),
    )(s, s0, h, wa, wb, wc, b)


def _pad_edges(ei):
    pad = EP - E
    src = jnp.concatenate([ei[0], jnp.zeros((pad,), jnp.int32)])
    dst = jnp.concatenate([ei[1], jnp.full((pad,), PAD_DST, jnp.int32)])
    return jnp.stack([src.reshape(NS, NBATCH, BATCH),
                      dst.reshape(NS, NBATCH, BATCH)])


def _deep_weights(wp, wn, bp, bn):
    blk = lambda w, i: w[i * H:(i + 1) * H]
    wa = jnp.concatenate([jnp.concatenate([blk(wp, 0), blk(wn, 3)], axis=1),
                          jnp.concatenate([blk(wp, 2), blk(wn, 1)], axis=1)], axis=0)
    wb = jnp.concatenate([jnp.concatenate([blk(wp, 3), blk(wn, 0)], axis=1),
                          jnp.concatenate([blk(wp, 1), blk(wn, 2)], axis=1)], axis=0)
    wc = jnp.concatenate(
        [jnp.concatenate([blk(wp, 4) + 0.5 * blk(wp, 6),
                          blk(wn, 5) + 0.5 * blk(wn, 6)], axis=1),
         jnp.concatenate([blk(wp, 5) + 0.5 * blk(wp, 6),
                          blk(wn, 4) + 0.5 * blk(wn, 6)], axis=1)], axis=0)
    b = jnp.concatenate([bp, bn]).reshape(1, 64)
    return wa, wb, wc, b


def kernel(x, W_pos_base, b_pos_base, W_neg_base, b_neg_base,
           W_pos_d1, b_pos_d1, W_pos_d2, b_pos_d2,
           W_neg_d1, b_neg_d1, W_neg_d2, b_neg_d2,
           pos_edge_index, neg_edge_index):
    edges = jnp.stack([_pad_edges(pos_edge_index), _pad_edges(neg_edge_index)])
    zeros80 = jnp.zeros((NPAD, W0), _f32)
    zeros64 = jnp.zeros((NPAD, 64), _f32)

    wtop = jnp.concatenate([W_pos_base[:D], W_neg_base[:D]], axis=1)
    wbot = jnp.concatenate([W_pos_base[D:], W_neg_base[D:]], axis=1)
    b0 = jnp.concatenate([b_pos_base, b_neg_base]).reshape(1, 64)

    table0, xself = _tc_proj(x, wtop, wbot)
    s0 = _sc_sums(table0, edges, zeros80, W0)
    h = _tc_layer0(s0, xself, b0)
    for (wp, bp, wn, bn) in ((W_pos_d1, b_pos_d1, W_neg_d1, b_neg_d1),
                             (W_pos_d2, b_pos_d2, W_neg_d2, b_neg_d2)):
        wa, wb, wc, b = _deep_weights(wp, wn, bp, bn)
        s = _sc_sums(h, edges, zeros64, 64)
        h = _tc_deep(s, s0, h, wa, wb, wc, b)
    return h


# speedup vs baseline: 3.4298x; 2.0084x over previous
import functools

import jax
import jax.numpy as jnp
from jax import lax
from jax.experimental import pallas as pl
from jax.experimental.pallas import tpu as pltpu
from jax.experimental.pallas import tpu_sc as plsc

N, E, D, H = 10000, 160000, 128, 32
NC, NS = 2, 16
BATCH = 128
EP = 163840
EDGES_PER_W = EP // NS
NBATCH = EDGES_PER_W // BATCH
NPAD = 10240
RPS = NPAD // NS
PAD_DST = N + 64
TCB = 10000
W0 = 72

_f32 = jnp.float32


def _sc_mesh():
    return plsc.VectorSubcoreMesh(core_axis_name="c", subcore_axis_name="s",
                                  num_cores=NC, num_subcores=NS)


def _sc_sums(table, edges, zeros, w):
    HB = NBATCH // 2

    @functools.partial(
        pl.kernel,
        out_type=jax.ShapeDtypeStruct((2, NPAD, w), _f32),
        mesh=_sc_mesh(),
        scratch_types=[
            pltpu.VMEM((HB, BATCH), jnp.int32),
            pltpu.VMEM((HB, BATCH), jnp.int32),
            pltpu.VMEM((BATCH, w), _f32),
            pltpu.VMEM((BATCH, w), _f32),
            pltpu.VMEM_SHARED((NPAD, w), _f32),
            pltpu.VMEM_SHARED((N, w), _f32),
            pltpu.SemaphoreType.DMA,
            pltpu.SemaphoreType.DMA,
        ],
        compiler_params=pltpu.CompilerParams(use_tc_tiling_on_sc=False),
    )
    def k(table_h, edges_h, zeros_h, out_s,
          src_half, dst_half, rows0, rows1, acc, tspm, semA, semB):
        cid = lax.axis_index("c")
        sid = lax.axis_index("s")
        r0 = sid * RPS
        pltpu.sync_copy(zeros_h.at[pl.ds(r0, RPS)], acc.at[pl.ds(r0, RPS)])
        t0 = sid * (N // NS)
        pltpu.sync_copy(table_h.at[pl.ds(t0, N // NS)], tspm.at[pl.ds(t0, N // NS)])
        plsc.subcore_barrier()

        def wait_for(buf, sem):
            pltpu.make_async_copy(tspm.at[src_half.at[0]], buf, sem).wait()

        for half in range(2):
            pltpu.sync_copy(edges_h.at[cid, 0, sid, pl.ds(half * HB, HB)], src_half)
            pltpu.sync_copy(edges_h.at[cid, 1, sid, pl.ds(half * HB, HB)], dst_half)
            pltpu.async_copy(tspm.at[src_half.at[0]], rows0, semA)

            def step(i, carry):
                j0 = 2 * i
                pltpu.async_copy(tspm.at[src_half.at[j0 + 1]], rows1, semB)
                wait_for(rows0, semA)
                pltpu.sync_copy(rows0, acc.at[dst_half.at[j0]], add=True)

                @pl.when(j0 + 2 < HB)
                def _():
                    pltpu.async_copy(tspm.at[src_half.at[j0 + 2]], rows0, semA)

                wait_for(rows1, semB)
                pltpu.sync_copy(rows1, acc.at[dst_half.at[j0 + 1]], add=True)
                return carry

            lax.fori_loop(0, HB // 2, step, 0)
        plsc.subcore_barrier()
        pltpu.sync_copy(acc.at[pl.ds(r0, RPS)], out_s.at[cid, pl.ds(r0, RPS)])

    return k(table, edges, zeros)


def _inv_counts(s0):
    cp = jnp.maximum(s0[0, :, W0 - 1:], 1.0)
    cn = jnp.maximum(s0[1, :, W0 - 1:], 1.0)
    return cp, cn


def _tc_proj(x, wtop, wbot):
    def body(x_ref, wt_ref, wb_ref, o1_ref, o2_ref):
        xb = x_ref[...]
        p = jnp.dot(xb, wt_ref[...], preferred_element_type=_f32)
        r = p.shape[0]
        o1_ref[...] = jnp.concatenate(
            [p, jnp.zeros((r, W0 - 65), _f32), jnp.ones((r, 1), _f32)], axis=1)
        o2_ref[...] = jnp.dot(xb, wb_ref[...], preferred_element_type=_f32)

    return pl.pallas_call(
        body,
        grid=(N // TCB,),
        in_specs=[pl.BlockSpec((TCB, D), lambda i: (i, 0)),
                  pl.BlockSpec((D, 64), lambda i: (0, 0)),
                  pl.BlockSpec((D, 64), lambda i: (0, 0))],
        out_specs=(pl.BlockSpec((TCB, W0), lambda i: (i, 0)),
                   pl.BlockSpec((TCB, 64), lambda i: (i, 0))),
        out_shape=(jax.ShapeDtypeStruct((N, W0), _f32),
                   jax.ShapeDtypeStruct((N, 64), _f32)),
    )(x, wtop, wbot)


def _tc_layer0(s0, xself, b0):
    def body(s_ref, p_ref, b_ref, o_ref):
        s0b = s_ref[...]
        cp, cn = _inv_counts(s0b)
        gp = s0b[0, :, :64] / cp
        gn = s0b[1, :, :64] / cn
        z = jnp.concatenate([gp[:, :32], gn[:, 32:]], axis=1) + p_ref[...] + b_ref[...]
        o_ref[...] = jnp.tanh(z)

    return pl.pallas_call(
        body,
        grid=(N // TCB,),
        in_specs=[pl.BlockSpec((2, TCB, W0), lambda i: (0, i, 0)),
                  pl.BlockSpec((TCB, 64), lambda i: (i, 0)),
                  pl.BlockSpec((1, 64), lambda i: (0, 0))],
        out_specs=pl.BlockSpec((TCB, 64), lambda i: (i, 0)),
        out_shape=jax.ShapeDtypeStruct((N, 64), _f32),
    )(s0, xself, b0)


def _tc_deep(s, s0, h, wa, wb, wc, b):
    def body(s_ref, s0_ref, h_ref, wa_ref, wb_ref, wc_ref, b_ref, o_ref):
        cp, cn = _inv_counts(s0_ref[...])
        sb = s_ref[...]
        gp = sb[0] / cp
        gn = sb[1] / cn
        z = (jnp.dot(gp, wa_ref[...], preferred_element_type=_f32)
             + jnp.dot(gn, wb_ref[...], preferred_element_type=_f32)
             + jnp.dot(h_ref[...], wc_ref[...], preferred_element_type=_f32)
             + b_ref[...])
        o_ref[...] = jnp.tanh(z)

    return pl.pallas_call(
        body,
        grid=(N // TCB,),
        in_specs=[pl.BlockSpec((2, TCB, 64), lambda i: (0, i, 0)),
                  pl.BlockSpec((2, TCB, W0), lambda i: (0, i, 0)),
                  pl.BlockSpec((TCB, 64), lambda i: (i, 0)),
                  pl.BlockSpec((64, 64), lambda i: (0, 0)),
                  pl.BlockSpec((64, 64), lambda i: (0, 0)),
                  pl.BlockSpec((64, 64), lambda i: (0, 0)),
                  pl.BlockSpec((1, 64), lambda i: (0, 0))],
        out_specs=pl.BlockSpec((TCB, 64), lambda i: (i, 0)),
        out_shape=jax.ShapeDtypeStruct((N, 64), _f32),
    )(s, s0, h, wa, wb, wc, b)


def _pad_edges(ei):
    pad = EP - E
    src = jnp.concatenate([ei[0], jnp.zeros((pad,), jnp.int32)])
    dst = jnp.concatenate([ei[1], jnp.full((pad,), PAD_DST, jnp.int32)])
    return jnp.stack([src.reshape(NS, NBATCH, BATCH),
                      dst.reshape(NS, NBATCH, BATCH)])


def _deep_weights(wp, wn, bp, bn):
    blk = lambda w, i: w[i * H:(i + 1) * H]
    wa = jnp.concatenate([jnp.concatenate([blk(wp, 0), blk(wn, 3)], axis=1),
                          jnp.concatenate([blk(wp, 2), blk(wn, 1)], axis=1)], axis=0)
    wb = jnp.concatenate([jnp.concatenate([blk(wp, 3), blk(wn, 0)], axis=1),
                          jnp.concatenate([blk(wp, 1), blk(wn, 2)], axis=1)], axis=0)
    wc = jnp.concatenate(
        [jnp.concatenate([blk(wp, 4) + 0.5 * blk(wp, 6),
                          blk(wn, 5) + 0.5 * blk(wn, 6)], axis=1),
         jnp.concatenate([blk(wp, 5) + 0.5 * blk(wp, 6),
                          blk(wn, 4) + 0.5 * blk(wn, 6)], axis=1)], axis=0)
    b = jnp.concatenate([bp, bn]).reshape(1, 64)
    return wa, wb, wc, b


def kernel(x, W_pos_base, b_pos_base, W_neg_base, b_neg_base,
           W_pos_d1, b_pos_d1, W_pos_d2, b_pos_d2,
           W_neg_d1, b_neg_d1, W_neg_d2, b_neg_d2,
           pos_edge_index, neg_edge_index):
    edges = jnp.stack([_pad_edges(pos_edge_index), _pad_edges(neg_edge_index)])
    zeros80 = jnp.zeros((NPAD, W0), _f32)
    zeros64 = jnp.zeros((NPAD, 64), _f32)

    wtop = jnp.concatenate([W_pos_base[:D], W_neg_base[:D]], axis=1)
    wbot = jnp.concatenate([W_pos_base[D:], W_neg_base[D:]], axis=1)
    b0 = jnp.concatenate([b_pos_base, b_neg_base]).reshape(1, 64)

    table0, xself = _tc_proj(x, wtop, wbot)
    s0 = _sc_sums(table0, edges, zeros80, W0)
    h = _tc_layer0(s0, xself, b0)
    for (wp, bp, wn, bn) in ((W_pos_d1, b_pos_d1, W_neg_d1, b_neg_d1),
                             (W_pos_d2, b_pos_d2, W_neg_d2, b_neg_d2)):
        wa, wb, wc, b = _deep_weights(wp, wn, bp, bn)
        s = _sc_sums(h, edges, zeros64, 64)
        h = _tc_deep(s, s0, h, wa, wb, wc, b)
    return h
